# Initial kernel scaffold; baseline (speedup 1.0000x reference)
import functools

import jax
import jax.numpy as jnp
from jax import lax
from jax.experimental import pallas as pl
from jax.experimental.pallas import tpu as pltpu
from jax.experimental.pallas import tpu_sc as plsc

NC = 2
NS = 16
NW = NC * NS
CHUNK = 128


def _sc_partial_call(idx2d, seg2d, embedding, g_per_w, sum_rows, d):
    mesh = plsc.VectorSubcoreMesh(
        core_axis_name="c", subcore_axis_name="s", num_cores=NC, num_subcores=NS
    )
    rows_per_tile = sum_rows // NS
    zchunks = rows_per_tile // CHUNK

    @functools.partial(
        pl.kernel,
        out_type=[
            jax.ShapeDtypeStruct((NC, sum_rows, d), jnp.float32),
            jax.ShapeDtypeStruct((NC, sum_rows, 16), jnp.float32),
        ],
        mesh=mesh,
        scratch_types=[
            pltpu.VMEM((g_per_w, CHUNK), jnp.int32),
            pltpu.VMEM((g_per_w, CHUNK), jnp.int32),
            pltpu.VMEM((CHUNK, d), jnp.float32),
            pltpu.VMEM((CHUNK, d), jnp.float32),
            pltpu.VMEM((CHUNK, 16), jnp.float32),
            pltpu.VMEM((CHUNK, 16), jnp.float32),
            pltpu.VMEM_SHARED((sum_rows, d), jnp.float32),
            pltpu.VMEM_SHARED((sum_rows, 16), jnp.float32),
            pltpu.SemaphoreType.DMA,
        ],
    )
    def sc_kernel(idx_hbm, seg_hbm, emb_hbm, psums_hbm, pcnts_hbm,
                  idx_v, seg_v, rows_v, zrow_v, zc_v, ones_v,
                  sums_sh, cnts_sh, sem):
        c = lax.axis_index("c")
        s = lax.axis_index("s")
        wid = s * NC + c

        def init_body(r, carry):
            for cc in range(d // 16):
                zrow_v[r, pl.ds(cc * 16, 16)] = jnp.zeros((16,), jnp.float32)
            zc_v[r, :] = jnp.zeros((16,), jnp.float32)
            ones_v[r, :] = jnp.full((16,), 1.0, jnp.float32)
            return carry

        lax.fori_loop(0, CHUNK, init_body, 0)

        for z in range(zchunks):
            base = s * rows_per_tile + z * CHUNK
            pltpu.sync_copy(zrow_v, sums_sh.at[pl.ds(base, CHUNK)])
            pltpu.sync_copy(zc_v, cnts_sh.at[pl.ds(base, CHUNK)])
        plsc.subcore_barrier()

        pltpu.sync_copy(idx_hbm.at[pl.ds(wid * g_per_w, g_per_w)], idx_v)
        pltpu.sync_copy(seg_hbm.at[pl.ds(wid * g_per_w, g_per_w)], seg_v)

        def chunk_body(j, carry):
            pltpu.async_copy(emb_hbm.at[idx_v.at[j]], rows_v, sem).wait()
            pltpu.sync_copy(rows_v, sums_sh.at[seg_v.at[j]], add=True)
            pltpu.sync_copy(ones_v, cnts_sh.at[seg_v.at[j]], add=True)
            return carry

        lax.fori_loop(0, g_per_w, chunk_body, 0)
        plsc.subcore_barrier()

        pltpu.sync_copy(
            sums_sh.at[pl.ds(s * rows_per_tile, rows_per_tile)],
            psums_hbm.at[c, pl.ds(s * rows_per_tile, rows_per_tile)],
        )
        pltpu.sync_copy(
            cnts_sh.at[pl.ds(s * rows_per_tile, rows_per_tile)],
            pcnts_hbm.at[c, pl.ds(s * rows_per_tile, rows_per_tile)],
        )

    return sc_kernel(idx2d, seg2d, embedding)


def _finalize_body(p0_ref, p1_ref, c0_ref, c1_ref, self_ref, out_ref):
    d = p0_ref.shape[1]
    cnt = c0_ref[:, 0:1] + c1_ref[:, 0:1]
    f1 = (p0_ref[...] + p1_ref[...]) / jnp.maximum(cnt, 1.0)
    out_ref[:, :d] = f1
    out_ref[:, d:] = self_ref[...] - f1


def kernel(embedding, self_feats, neighbor_idx, segment_ids):
    n, d = embedding.shape
    b = self_feats.shape[0]
    e = neighbor_idx.shape[0]

    g_per_w = -(-e // (NW * CHUNK))
    ep = g_per_w * NW * CHUNK
    pad = ep - e
    idx_p = jnp.concatenate(
        [neighbor_idx, jnp.zeros((pad,), jnp.int32)]).reshape(ep // CHUNK, CHUNK)
    seg_p = jnp.concatenate(
        [segment_ids, jnp.full((pad,), b, jnp.int32)]).reshape(ep // CHUNK, CHUNK)

    sum_rows = -(-(b + 1) // (NS * CHUNK)) * NS * CHUNK

    psums, pcnts = _sc_partial_call(idx_p, seg_p, embedding, g_per_w, sum_rows, d)

    bs = 400
    out = pl.pallas_call(
        _finalize_body,
        grid=(b // bs,),
        in_specs=[
            pl.BlockSpec((bs, d), lambda i: (i, 0)),
            pl.BlockSpec((bs, d), lambda i: (i, 0)),
            pl.BlockSpec((bs, 16), lambda i: (i, 0)),
            pl.BlockSpec((bs, 16), lambda i: (i, 0)),
            pl.BlockSpec((bs, d), lambda i: (i, 0)),
        ],
        out_specs=pl.BlockSpec((bs, 2 * d), lambda i: (i, 0)),
        out_shape=jax.ShapeDtypeStruct((b, 2 * d), jnp.float32),
    )(psums[0, :b], psums[1, :b], pcnts[0, :b], pcnts[1, :b], self_feats)
    return out


# speedup vs baseline: 4.0835x; 4.0835x over previous
import functools

import jax
import jax.numpy as jnp
from jax import lax
from jax.experimental import pallas as pl
from jax.experimental.pallas import tpu as pltpu
from jax.experimental.pallas import tpu_sc as plsc

NC = 2
NS = 16
CHUNK = 128


def _sc_partial_call(idx2d, seg2d, emb2, g_per_w, sum_rows, dh):
    mesh = plsc.VectorSubcoreMesh(
        core_axis_name="c", subcore_axis_name="s", num_cores=NC, num_subcores=NS
    )
    rows_per_tile = sum_rows // NS
    zchunks = rows_per_tile // CHUNK

    @functools.partial(
        pl.kernel,
        out_type=[
            jax.ShapeDtypeStruct((NC, sum_rows, dh), jnp.float32),
            jax.ShapeDtypeStruct((NC, sum_rows, 16), jnp.float32),
        ],
        mesh=mesh,
        compiler_params=pltpu.CompilerParams(use_tc_tiling_on_sc=False),
        scratch_types=[
            pltpu.VMEM((g_per_w, CHUNK), jnp.int32),
            pltpu.VMEM((g_per_w, CHUNK), jnp.int32),
            pltpu.VMEM((CHUNK, dh), jnp.float32),
            pltpu.VMEM((CHUNK, dh), jnp.float32),
            pltpu.VMEM((CHUNK, 16), jnp.float32),
            pltpu.VMEM((CHUNK, 16), jnp.float32),
            pltpu.VMEM_SHARED((sum_rows, dh), jnp.float32),
            pltpu.VMEM_SHARED((sum_rows, 16), jnp.float32),
            pltpu.SemaphoreType.DMA,
        ],
    )
    def sc_kernel(idx_hbm, seg_hbm, emb_hbm, psums_hbm, pcnts_hbm,
                  idx_v, seg_v, rows_v, zrow_v, zc_v, ones_v,
                  sums_sh, cnts_sh, sem):
        c = lax.axis_index("c")
        s = lax.axis_index("s")

        def init_body(r, carry):
            for cc in range(dh // 16):
                zrow_v[r, pl.ds(cc * 16, 16)] = jnp.zeros((16,), jnp.float32)
            zc_v[r, :] = jnp.zeros((16,), jnp.float32)
            ones_v[r, :] = jnp.full((16,), 1.0, jnp.float32)
            return carry

        lax.fori_loop(0, CHUNK, init_body, 0)

        for z in range(zchunks):
            base = s * rows_per_tile + z * CHUNK
            pltpu.sync_copy(zrow_v, sums_sh.at[pl.ds(base, CHUNK)])
            pltpu.sync_copy(zc_v, cnts_sh.at[pl.ds(base, CHUNK)])
        plsc.subcore_barrier()

        pltpu.sync_copy(idx_hbm.at[pl.ds(s * g_per_w, g_per_w)], idx_v)
        pltpu.sync_copy(seg_hbm.at[pl.ds(s * g_per_w, g_per_w)], seg_v)
        c_vec = jnp.full((16,), c, jnp.int32)

        def adj_body(r, carry):
            for cc in range(CHUNK // 16):
                sl = pl.ds(cc * 16, 16)
                idx_v[r, sl] = idx_v[r, sl] + c_vec
            return carry

        lax.fori_loop(0, g_per_w, adj_body, 0)

        def chunk_body(j, carry):
            pltpu.async_copy(emb_hbm.at[idx_v.at[j]], rows_v, sem).wait()
            pltpu.sync_copy(rows_v, sums_sh.at[seg_v.at[j]], add=True)
            pltpu.sync_copy(ones_v, cnts_sh.at[seg_v.at[j]], add=True)
            return carry

        lax.fori_loop(0, g_per_w, chunk_body, 0)
        plsc.subcore_barrier()

        pltpu.sync_copy(
            sums_sh.at[pl.ds(s * rows_per_tile, rows_per_tile)],
            psums_hbm.at[c, pl.ds(s * rows_per_tile, rows_per_tile)],
        )
        pltpu.sync_copy(
            cnts_sh.at[pl.ds(s * rows_per_tile, rows_per_tile)],
            pcnts_hbm.at[c, pl.ds(s * rows_per_tile, rows_per_tile)],
        )

    return sc_kernel(idx2d, seg2d, emb2)


def _finalize_body(p0_ref, p1_ref, c0_ref, self_ref, out_ref):
    dh = p0_ref.shape[1]
    inv = 1.0 / jnp.maximum(c0_ref[:, 0:1], 1.0)
    f10 = p0_ref[...] * inv
    f11 = p1_ref[...] * inv
    out_ref[:, :dh] = f10
    out_ref[:, dh:2 * dh] = f11
    out_ref[:, 2 * dh:3 * dh] = self_ref[:, :dh] - f10
    out_ref[:, 3 * dh:] = self_ref[:, dh:] - f11


def kernel(embedding, self_feats, neighbor_idx, segment_ids):
    n, d = embedding.shape
    dh = d // 2
    b = self_feats.shape[0]
    e = neighbor_idx.shape[0]

    g_per_w = -(-(-(-e // (NS * CHUNK))) // 8) * 8
    ep = g_per_w * NS * CHUNK
    pad = ep - e
    idx_p = jnp.concatenate(
        [neighbor_idx * 2, jnp.zeros((pad,), jnp.int32)]).reshape(ep // CHUNK, CHUNK)
    seg_p = jnp.concatenate(
        [segment_ids, jnp.full((pad,), b, jnp.int32)]).reshape(ep // CHUNK, CHUNK)
    emb2 = embedding.reshape(n * 2, dh)

    sum_rows = -(-(b + 1) // (NS * CHUNK)) * NS * CHUNK

    psums, pcnts = _sc_partial_call(idx_p, seg_p, emb2, g_per_w, sum_rows, dh)

    bs = 400
    out = pl.pallas_call(
        _finalize_body,
        grid=(b // bs,),
        in_specs=[
            pl.BlockSpec((bs, dh), lambda i: (i, 0)),
            pl.BlockSpec((bs, dh), lambda i: (i, 0)),
            pl.BlockSpec((bs, 16), lambda i: (i, 0)),
            pl.BlockSpec((bs, d), lambda i: (i, 0)),
        ],
        out_specs=pl.BlockSpec((bs, 2 * d), lambda i: (i, 0)),
        out_shape=jax.ShapeDtypeStruct((b, 2 * d), jnp.float32),
    )(psums[0, :b], psums[1, :b], pcnts[0, :b], self_feats)
    return out


# speedup vs baseline: 4.7745x; 1.1692x over previous
import functools

import jax
import jax.numpy as jnp
from jax import lax
from jax.experimental import pallas as pl
from jax.experimental.pallas import tpu as pltpu
from jax.experimental.pallas import tpu_sc as plsc

NC = 2
NS = 16
CHUNK = 128


def _sc_partial_call(idx2d, seg2d, emb2, g_per_w, sum_rows, dh):
    mesh = plsc.VectorSubcoreMesh(
        core_axis_name="c", subcore_axis_name="s", num_cores=NC, num_subcores=NS
    )
    rows_per_tile = sum_rows // NS
    zchunks = rows_per_tile // CHUNK

    @functools.partial(
        pl.kernel,
        out_type=[
            jax.ShapeDtypeStruct((NC, sum_rows, dh), jnp.float32),
            jax.ShapeDtypeStruct((NC, sum_rows, 16), jnp.float32),
        ],
        mesh=mesh,
        compiler_params=pltpu.CompilerParams(use_tc_tiling_on_sc=False),
        scratch_types=[
            pltpu.VMEM((g_per_w, CHUNK), jnp.int32),
            pltpu.VMEM((g_per_w, CHUNK), jnp.int32),
            pltpu.VMEM((4, CHUNK, dh), jnp.float32),
            pltpu.VMEM((CHUNK, 16), jnp.float32),
            pltpu.VMEM((CHUNK, 16), jnp.float32),
            pltpu.VMEM_SHARED((sum_rows, dh), jnp.float32),
            pltpu.VMEM_SHARED((sum_rows, 16), jnp.float32),
            pltpu.SemaphoreType.DMA,
            pltpu.SemaphoreType.DMA,
            pltpu.SemaphoreType.DMA,
        ],
    )
    def sc_kernel(idx_hbm, seg_hbm, emb_hbm, psums_hbm, pcnts_hbm,
                  idx_v, seg_v, rows_v, zc_v, ones_v,
                  sums_sh, cnts_sh, sem_g, sem_s, sem_c):
        c = lax.axis_index("c")
        s = lax.axis_index("s")

        def init_body(r, carry):
            for cc in range(dh // 16):
                rows_v[0, r, pl.ds(cc * 16, 16)] = jnp.zeros((16,), jnp.float32)
            zc_v[r, :] = jnp.zeros((16,), jnp.float32)
            ones_v[r, :] = jnp.full((16,), 1.0, jnp.float32)
            return carry

        lax.fori_loop(0, CHUNK, init_body, 0)

        for z in range(zchunks):
            base = s * rows_per_tile + z * CHUNK
            pltpu.sync_copy(rows_v.at[0], sums_sh.at[pl.ds(base, CHUNK)])
            pltpu.sync_copy(zc_v, cnts_sh.at[pl.ds(base, CHUNK)])
        plsc.subcore_barrier()

        pltpu.sync_copy(idx_hbm.at[pl.ds(s * g_per_w, g_per_w)], idx_v)
        pltpu.sync_copy(seg_hbm.at[pl.ds(s * g_per_w, g_per_w)], seg_v)
        c_vec = jnp.full((16,), c, jnp.int32)

        def adj_body(r, carry):
            for cc in range(CHUNK // 16):
                sl = pl.ds(cc * 16, 16)
                idx_v[r, sl] = idx_v[r, sl] + c_vec
            return carry

        lax.fori_loop(0, g_per_w, adj_body, 0)

        def fire_g(j, k):
            pltpu.async_copy(emb_hbm.at[idx_v.at[j]], rows_v.at[k], sem_g)

        def wait_g(j, k):
            pltpu.make_async_copy(emb_hbm.at[idx_v.at[j]], rows_v.at[k],
                                  sem_g).wait()

        def fire_s(j, k):
            pltpu.async_copy(rows_v.at[k], sums_sh.at[seg_v.at[j]], sem_s,
                             add=True)
            pltpu.async_copy(ones_v, cnts_sh.at[seg_v.at[j]], sem_c, add=True)

        def wait_s(j, k):
            pltpu.make_async_copy(rows_v.at[k], sums_sh.at[seg_v.at[j]],
                                  sem_s).wait()
            pltpu.make_async_copy(ones_v, cnts_sh.at[seg_v.at[j]],
                                  sem_c).wait()

        def quad(i, drain_prev, fire_next):
            j0 = 4 * i
            wait_g(j0, 0)
            wait_g(j0 + 1, 1)
            if drain_prev:
                wait_s(j0 - 2, 2)
                wait_s(j0 - 1, 3)
            fire_g(j0 + 2, 2)
            fire_g(j0 + 3, 3)
            fire_s(j0, 0)
            fire_s(j0 + 1, 1)
            wait_g(j0 + 2, 2)
            wait_g(j0 + 3, 3)
            wait_s(j0, 0)
            wait_s(j0 + 1, 1)
            fire_s(j0 + 2, 2)
            fire_s(j0 + 3, 3)
            if fire_next:
                fire_g(j0 + 4, 0)
                fire_g(j0 + 5, 1)

        nb = g_per_w // 4
        fire_g(0, 0)
        fire_g(1, 1)
        quad(0, False, True)

        def chunk_body(i, carry):
            quad(i, True, True)
            return carry

        lax.fori_loop(1, nb - 1, chunk_body, 0)
        quad(nb - 1, True, False)
        wait_s(g_per_w - 2, 2)
        wait_s(g_per_w - 1, 3)
        plsc.subcore_barrier()


        pltpu.sync_copy(
            sums_sh.at[pl.ds(s * rows_per_tile, rows_per_tile)],
            psums_hbm.at[c, pl.ds(s * rows_per_tile, rows_per_tile)],
        )
        pltpu.sync_copy(
            cnts_sh.at[pl.ds(s * rows_per_tile, rows_per_tile)],
            pcnts_hbm.at[c, pl.ds(s * rows_per_tile, rows_per_tile)],
        )

    return sc_kernel(idx2d, seg2d, emb2)


def _finalize_body(p0_ref, p1_ref, c0_ref, self_ref, out_ref):
    dh = p0_ref.shape[1]
    inv = 1.0 / jnp.maximum(c0_ref[:, 0:1], 1.0)
    f10 = p0_ref[...] * inv
    f11 = p1_ref[...] * inv
    out_ref[:, :dh] = f10
    out_ref[:, dh:2 * dh] = f11
    out_ref[:, 2 * dh:3 * dh] = self_ref[:, :dh] - f10
    out_ref[:, 3 * dh:] = self_ref[:, dh:] - f11


def kernel(embedding, self_feats, neighbor_idx, segment_ids):
    n, d = embedding.shape
    dh = d // 2
    b = self_feats.shape[0]
    e = neighbor_idx.shape[0]

    g_per_w = -(-(-(-e // (NS * CHUNK))) // 8) * 8
    ep = g_per_w * NS * CHUNK
    pad = ep - e
    idx_p = jnp.concatenate(
        [neighbor_idx * 2, jnp.zeros((pad,), jnp.int32)]).reshape(ep // CHUNK, CHUNK)
    seg_p = jnp.concatenate(
        [segment_ids, jnp.full((pad,), b, jnp.int32)]).reshape(ep // CHUNK, CHUNK)
    emb2 = embedding.reshape(n * 2, dh)

    sum_rows = -(-(b + 1) // (NS * CHUNK)) * NS * CHUNK

    psums, pcnts = _sc_partial_call(idx_p, seg_p, emb2, g_per_w, sum_rows, dh)

    bs = 400
    out = pl.pallas_call(
        _finalize_body,
        grid=(b // bs,),
        in_specs=[
            pl.BlockSpec((bs, dh), lambda i: (i, 0)),
            pl.BlockSpec((bs, dh), lambda i: (i, 0)),
            pl.BlockSpec((bs, 16), lambda i: (i, 0)),
            pl.BlockSpec((bs, d), lambda i: (i, 0)),
        ],
        out_specs=pl.BlockSpec((bs, 2 * d), lambda i: (i, 0)),
        out_shape=jax.ShapeDtypeStruct((b, 2 * d), jnp.float32),
    )(psums[0, :b], psums[1, :b], pcnts[0, :b], self_feats)
    return out


# speedup vs baseline: 7.1760x; 1.5030x over previous
import functools

import jax
import jax.numpy as jnp
from jax import lax
from jax.experimental import pallas as pl
from jax.experimental.pallas import tpu as pltpu
from jax.experimental.pallas import tpu_sc as plsc

NC = 2
NS = 16
CHUNK = 128
QUAD = 4


def _sc_partial_call(idx2d, seg2d, embT, g_per_w, sum_rows, tab_rows, dh):
    mesh = plsc.VectorSubcoreMesh(
        core_axis_name="c", subcore_axis_name="s", num_cores=NC, num_subcores=NS
    )
    rows_per_tile = sum_rows // NS
    zchunks = rows_per_tile // CHUNK
    tab_per_tile = tab_rows // NS

    @functools.partial(
        pl.kernel,
        out_type=[
            jax.ShapeDtypeStruct((NC, sum_rows, dh), jnp.float32),
            jax.ShapeDtypeStruct((NC, sum_rows, 16), jnp.float32),
        ],
        mesh=mesh,
        compiler_params=pltpu.CompilerParams(use_tc_tiling_on_sc=False),
        scratch_types=[
            pltpu.VMEM((2, QUAD, CHUNK), jnp.int32),
            pltpu.VMEM((2, QUAD, CHUNK), jnp.int32),
            pltpu.VMEM((4, CHUNK, dh), jnp.float32),
            pltpu.VMEM((CHUNK, 16), jnp.float32),
            pltpu.VMEM((CHUNK, 16), jnp.float32),
            pltpu.VMEM_SHARED((tab_rows, dh), jnp.float32),
            pltpu.VMEM_SHARED((sum_rows, dh), jnp.float32),
            pltpu.VMEM_SHARED((sum_rows, 16), jnp.float32),
            pltpu.SemaphoreType.DMA,
            pltpu.SemaphoreType.DMA,
            pltpu.SemaphoreType.DMA,
            pltpu.SemaphoreType.DMA,
        ],
    )
    def sc_kernel(idx_hbm, seg_hbm, embT_hbm, psums_hbm, pcnts_hbm,
                  idx_v, seg_v, rows_v, zc_v, ones_v,
                  table_sh, sums_sh, cnts_sh, sem_g, sem_s, sem_c, sem_i):
        c = lax.axis_index("c")
        s = lax.axis_index("s")

        def init_body(r, carry):
            for cc in range(dh // 16):
                rows_v[0, r, pl.ds(cc * 16, 16)] = jnp.zeros((16,), jnp.float32)
            zc_v[r, :] = jnp.zeros((16,), jnp.float32)
            ones_v[r, :] = jnp.full((16,), 1.0, jnp.float32)
            return carry

        lax.fori_loop(0, CHUNK, init_body, 0)

        pltpu.sync_copy(
            embT_hbm.at[c, pl.ds(s * tab_per_tile, tab_per_tile)],
            table_sh.at[pl.ds(s * tab_per_tile, tab_per_tile)],
        )
        for z in range(zchunks):
            base = s * rows_per_tile + z * CHUNK
            pltpu.sync_copy(rows_v.at[0], sums_sh.at[pl.ds(base, CHUNK)])
            pltpu.sync_copy(zc_v, cnts_sh.at[pl.ds(base, CHUNK)])
        plsc.subcore_barrier()

        def fire_blk(q, slot):
            base = s * g_per_w + q * QUAD
            pltpu.async_copy(idx_hbm.at[pl.ds(base, QUAD)], idx_v.at[slot],
                             sem_i)
            pltpu.async_copy(seg_hbm.at[pl.ds(base, QUAD)], seg_v.at[slot],
                             sem_i)

        def wait_blk(q, slot):
            base = s * g_per_w + q * QUAD
            pltpu.make_async_copy(idx_hbm.at[pl.ds(base, QUAD)],
                                  idx_v.at[slot], sem_i).wait()
            pltpu.make_async_copy(seg_hbm.at[pl.ds(base, QUAD)],
                                  seg_v.at[slot], sem_i).wait()

        def fire_g(slot, jj, k):
            pltpu.async_copy(table_sh.at[idx_v.at[slot, jj]], rows_v.at[k],
                             sem_g)

        def wait_g(slot, jj, k):
            pltpu.make_async_copy(table_sh.at[idx_v.at[slot, jj]],
                                  rows_v.at[k], sem_g).wait()

        def fire_s(slot, jj, k):
            pltpu.async_copy(rows_v.at[k], sums_sh.at[seg_v.at[slot, jj]],
                             sem_s, add=True)
            pltpu.async_copy(ones_v, cnts_sh.at[seg_v.at[slot, jj]], sem_c,
                             add=True)

        def wait_s(slot, jj, k):
            pltpu.make_async_copy(rows_v.at[k], sums_sh.at[seg_v.at[slot, jj]],
                                  sem_s).wait()
            pltpu.make_async_copy(ones_v, cnts_sh.at[seg_v.at[slot, jj]],
                                  sem_c).wait()

        def quad_body(q, sl, drain_prev, fire_next):
            nsl = 1 - sl
            wait_g(sl, 0, 0)
            wait_g(sl, 1, 1)
            if drain_prev:
                wait_s(nsl, 2, 2)
                wait_s(nsl, 3, 3)
            if fire_next:
                fire_blk(q + 1, nsl)
            fire_g(sl, 2, 2)
            fire_g(sl, 3, 3)
            fire_s(sl, 0, 0)
            fire_s(sl, 1, 1)
            wait_g(sl, 2, 2)
            wait_g(sl, 3, 3)
            wait_s(sl, 0, 0)
            wait_s(sl, 1, 1)
            fire_s(sl, 2, 2)
            fire_s(sl, 3, 3)
            if fire_next:
                wait_blk(q + 1, nsl)
                fire_g(nsl, 0, 0)
                fire_g(nsl, 1, 1)

        nq = g_per_w // QUAD
        fire_blk(0, 0)
        wait_blk(0, 0)
        fire_g(0, 0, 0)
        fire_g(0, 1, 1)
        quad_body(0, 0, False, True)

        def loop_body(q, carry):
            quad_body(q, q % 2, True, True)
            return carry

        lax.fori_loop(1, nq - 1, loop_body, 0)
        quad_body(nq - 1, (nq - 1) % 2, True, False)
        wait_s((nq - 1) % 2, 2, 2)
        wait_s((nq - 1) % 2, 3, 3)
        plsc.subcore_barrier()

        pltpu.sync_copy(
            sums_sh.at[pl.ds(s * rows_per_tile, rows_per_tile)],
            psums_hbm.at[c, pl.ds(s * rows_per_tile, rows_per_tile)],
        )
        pltpu.sync_copy(
            cnts_sh.at[pl.ds(s * rows_per_tile, rows_per_tile)],
            pcnts_hbm.at[c, pl.ds(s * rows_per_tile, rows_per_tile)],
        )

    return sc_kernel(idx2d, seg2d, embT)


def _finalize_body(p0_ref, p1_ref, c0_ref, self_ref, out_ref):
    dh = p0_ref.shape[1]
    inv = 1.0 / jnp.maximum(c0_ref[:, 0:1], 1.0)
    f10 = p0_ref[...] * inv
    f11 = p1_ref[...] * inv
    out_ref[:, :dh] = f10
    out_ref[:, dh:2 * dh] = f11
    out_ref[:, 2 * dh:3 * dh] = self_ref[:, :dh] - f10
    out_ref[:, 3 * dh:] = self_ref[:, dh:] - f11


def kernel(embedding, self_feats, neighbor_idx, segment_ids):
    n, d = embedding.shape
    dh = d // 2
    b = self_feats.shape[0]
    e = neighbor_idx.shape[0]

    g_per_w = -(-(-(-e // (NS * CHUNK))) // 8) * 8
    ep = g_per_w * NS * CHUNK
    pad = ep - e
    idx_p = jnp.concatenate(
        [neighbor_idx, jnp.zeros((pad,), jnp.int32)]).reshape(ep // CHUNK, CHUNK)
    seg_p = jnp.concatenate(
        [segment_ids, jnp.full((pad,), b, jnp.int32)]).reshape(ep // CHUNK, CHUNK)

    tab_rows = -(-n // (NS * 8)) * NS * 8
    tpad = tab_rows - n
    embT = jnp.stack([
        jnp.pad(embedding[:, :dh], ((0, tpad), (0, 0))),
        jnp.pad(embedding[:, dh:], ((0, tpad), (0, 0))),
    ])

    sum_rows = -(-(b + 1) // (NS * CHUNK)) * NS * CHUNK

    psums, pcnts = _sc_partial_call(
        idx_p, seg_p, embT, g_per_w, sum_rows, tab_rows, dh)

    bs = 400
    out = pl.pallas_call(
        _finalize_body,
        grid=(b // bs,),
        in_specs=[
            pl.BlockSpec((bs, dh), lambda i: (i, 0)),
            pl.BlockSpec((bs, dh), lambda i: (i, 0)),
            pl.BlockSpec((bs, 16), lambda i: (i, 0)),
            pl.BlockSpec((bs, d), lambda i: (i, 0)),
        ],
        out_specs=pl.BlockSpec((bs, 2 * d), lambda i: (i, 0)),
        out_shape=jax.ShapeDtypeStruct((b, 2 * d), jnp.float32),
    )(psums[0, :b], psums[1, :b], pcnts[0, :b], self_feats)
    return out


# speedup vs baseline: 7.2944x; 1.0165x over previous
import functools

import jax
import jax.numpy as jnp
from jax import lax
from jax.experimental import pallas as pl
from jax.experimental.pallas import tpu as pltpu
from jax.experimental.pallas import tpu_sc as plsc

NC = 2
NS = 16
CHUNK = 128
QUAD = 4


def _sc_call(idx2d, seg2d, emb, selfp, g_per_w, sum_rows, dh):
    n = emb.shape[0]
    mesh = plsc.VectorSubcoreMesh(
        core_axis_name="c", subcore_axis_name="s", num_cores=NC, num_subcores=NS
    )
    rows_per_tile = sum_rows // NS
    zchunks = rows_per_tile // CHUNK
    tab_main = (n // (NS * 8)) * 8
    tab_rem = n - NS * tab_main
    tab_rows = -(-n // 8) * 8

    @functools.partial(
        pl.kernel,
        out_type=jax.ShapeDtypeStruct((4, sum_rows, dh), jnp.float32),
        mesh=mesh,
        compiler_params=pltpu.CompilerParams(use_tc_tiling_on_sc=False),
        scratch_types=[
            pltpu.VMEM((2, QUAD, CHUNK), jnp.int32),
            pltpu.VMEM((2, QUAD, CHUNK), jnp.int32),
            pltpu.VMEM((4, CHUNK, dh), jnp.float32),
            pltpu.VMEM((CHUNK, 16), jnp.float32),
            pltpu.VMEM((CHUNK, 16), jnp.float32),
            pltpu.VMEM_SHARED((tab_rows, dh), jnp.float32),
            pltpu.VMEM_SHARED((sum_rows, dh), jnp.float32),
            pltpu.VMEM_SHARED((sum_rows, 16), jnp.float32),
            pltpu.SemaphoreType.DMA,
            pltpu.SemaphoreType.DMA,
            pltpu.SemaphoreType.DMA,
            pltpu.SemaphoreType.DMA,
        ],
    )
    def sc_kernel(idx_hbm, seg_hbm, emb_hbm, selfp_hbm, out_hbm,
                  idx_v, seg_v, rows_v, zc_v, ones_v,
                  table_sh, sums_sh, cnts_sh, sem_g, sem_s, sem_c, sem_i):
        c = lax.axis_index("c")
        s = lax.axis_index("s")

        def init_body(r, carry):
            for cc in range(dh // 16):
                rows_v[0, r, pl.ds(cc * 16, 16)] = jnp.zeros((16,), jnp.float32)
            zc_v[r, :] = jnp.zeros((16,), jnp.float32)
            ones_v[r, :] = jnp.full((16,), 1.0, jnp.float32)
            return carry

        lax.fori_loop(0, CHUNK, init_body, 0)

        pltpu.sync_copy(
            emb_hbm.at[pl.ds(s * tab_main, tab_main), pl.ds(c * dh, dh)],
            table_sh.at[pl.ds(s * tab_main, tab_main)],
        )
        if tab_rem:
            @pl.when(s == 0)
            def _():
                pltpu.sync_copy(
                    emb_hbm.at[pl.ds(NS * tab_main, tab_rem),
                               pl.ds(c * dh, dh)],
                    table_sh.at[pl.ds(NS * tab_main, tab_rem)],
                )
        for z in range(zchunks):
            base = s * rows_per_tile + z * CHUNK
            pltpu.sync_copy(rows_v.at[0], sums_sh.at[pl.ds(base, CHUNK)])
            pltpu.sync_copy(zc_v, cnts_sh.at[pl.ds(base, CHUNK)])
        plsc.subcore_barrier()

        def fire_blk(q, slot):
            base = s * g_per_w + q * QUAD
            pltpu.async_copy(idx_hbm.at[pl.ds(base, QUAD)], idx_v.at[slot],
                             sem_i)
            pltpu.async_copy(seg_hbm.at[pl.ds(base, QUAD)], seg_v.at[slot],
                             sem_i)

        def wait_blk(q, slot):
            base = s * g_per_w + q * QUAD
            pltpu.make_async_copy(idx_hbm.at[pl.ds(base, QUAD)],
                                  idx_v.at[slot], sem_i).wait()
            pltpu.make_async_copy(seg_hbm.at[pl.ds(base, QUAD)],
                                  seg_v.at[slot], sem_i).wait()

        def fire_g(slot, jj, k):
            pltpu.async_copy(table_sh.at[idx_v.at[slot, jj]], rows_v.at[k],
                             sem_g)

        def wait_g(slot, jj, k):
            pltpu.make_async_copy(table_sh.at[idx_v.at[slot, jj]],
                                  rows_v.at[k], sem_g).wait()

        def fire_s(slot, jj, k):
            pltpu.async_copy(rows_v.at[k], sums_sh.at[seg_v.at[slot, jj]],
                             sem_s, add=True)
            pltpu.async_copy(ones_v, cnts_sh.at[seg_v.at[slot, jj]], sem_c,
                             add=True)

        def wait_s(slot, jj, k):
            pltpu.make_async_copy(rows_v.at[k], sums_sh.at[seg_v.at[slot, jj]],
                                  sem_s).wait()
            pltpu.make_async_copy(ones_v, cnts_sh.at[seg_v.at[slot, jj]],
                                  sem_c).wait()

        def quad_body(q, sl, drain_prev, fire_next):
            nsl = 1 - sl
            wait_g(sl, 0, 0)
            wait_g(sl, 1, 1)
            if drain_prev:
                wait_s(nsl, 2, 2)
                wait_s(nsl, 3, 3)
            if fire_next:
                fire_blk(q + 1, nsl)
            fire_g(sl, 2, 2)
            fire_g(sl, 3, 3)
            fire_s(sl, 0, 0)
            fire_s(sl, 1, 1)
            wait_g(sl, 2, 2)
            wait_g(sl, 3, 3)
            wait_s(sl, 0, 0)
            wait_s(sl, 1, 1)
            fire_s(sl, 2, 2)
            fire_s(sl, 3, 3)
            if fire_next:
                wait_blk(q + 1, nsl)
                fire_g(nsl, 0, 0)
                fire_g(nsl, 1, 1)

        nq = g_per_w // QUAD
        fire_blk(0, 0)
        wait_blk(0, 0)
        fire_g(0, 0, 0)
        fire_g(0, 1, 1)
        quad_body(0, 0, False, True)

        def loop_body(q, carry):
            quad_body(q, q % 2, True, True)
            return carry

        lax.fori_loop(1, nq - 1, loop_body, 0)
        quad_body(nq - 1, (nq - 1) % 2, True, False)
        wait_s((nq - 1) % 2, 2, 2)
        wait_s((nq - 1) % 2, 3, 3)
        plsc.subcore_barrier()

        for z in range(zchunks):
            base = s * rows_per_tile + z * CHUNK
            pltpu.sync_copy(sums_sh.at[pl.ds(base, CHUNK)], rows_v.at[0])
            pltpu.sync_copy(cnts_sh.at[pl.ds(base, CHUNK)], zc_v)
            pltpu.sync_copy(
                selfp_hbm.at[pl.ds(base, CHUNK), pl.ds(c * dh, dh)],
                rows_v.at[1],
            )

            def fin_body(r, carry):
                iv = jnp.full((16,), 1.0, jnp.float32) / jnp.maximum(
                    zc_v[r, :], 1.0)
                for q in range(dh // 16):
                    sl = pl.ds(q * 16, 16)
                    f1 = rows_v[0, r, sl] * iv
                    rows_v[2, r, sl] = f1
                    rows_v[3, r, sl] = rows_v[1, r, sl] - f1
                return carry

            lax.fori_loop(0, CHUNK, fin_body, 0)
            pltpu.sync_copy(rows_v.at[2], out_hbm.at[c, pl.ds(base, CHUNK)])
            pltpu.sync_copy(rows_v.at[3],
                            out_hbm.at[2 + c, pl.ds(base, CHUNK)])

    return sc_kernel(idx2d, seg2d, emb, selfp)


def _assemble_body(o0_ref, o1_ref, o2_ref, o3_ref, out_ref):
    dh = o0_ref.shape[2]
    out_ref[:, :dh] = o0_ref[0]
    out_ref[:, dh:2 * dh] = o1_ref[0]
    out_ref[:, 2 * dh:3 * dh] = o2_ref[0]
    out_ref[:, 3 * dh:] = o3_ref[0]


def kernel(embedding, self_feats, neighbor_idx, segment_ids):
    n, d = embedding.shape
    dh = d // 2
    b = self_feats.shape[0]
    e = neighbor_idx.shape[0]

    g_per_w = -(-(-(-e // (NS * CHUNK))) // 8) * 8
    ep = g_per_w * NS * CHUNK
    pad = ep - e
    idx_p = jnp.concatenate(
        [neighbor_idx, jnp.zeros((pad,), jnp.int32)]).reshape(ep // CHUNK, CHUNK)
    seg_p = jnp.concatenate(
        [segment_ids, jnp.full((pad,), b, jnp.int32)]).reshape(ep // CHUNK, CHUNK)

    sum_rows = -(-(b + 1) // (NS * CHUNK)) * NS * CHUNK
    selfp = jnp.pad(self_feats, ((0, sum_rows - b), (0, 0)))

    planes = _sc_call(idx_p, seg_p, embedding, selfp, g_per_w, sum_rows, dh)

    bs = 400
    spec = lambda p: pl.BlockSpec((1, bs, dh), lambda i, p=p: (p, i, 0))
    out = pl.pallas_call(
        _assemble_body,
        grid=(b // bs,),
        in_specs=[spec(0), spec(1), spec(2), spec(3)],
        out_specs=pl.BlockSpec((bs, 2 * d), lambda i: (i, 0)),
        out_shape=jax.ShapeDtypeStruct((b, 2 * d), jnp.float32),
    )(planes, planes, planes, planes)
    return out
